# baseline (speedup 1.0000x reference)
import functools

import jax
import jax.numpy as jnp
from jax import lax
from jax.experimental import pallas as pl
from jax.experimental.pallas import tpu as pltpu
from jax.experimental.pallas import tpu_sc as plsc

N = 10000
E = 320000
D = 128
H = 8
DH = 16
HID = 128
EPS = 1e-5

NC = 2
NS = 16
NW = NC * NS
EPT = E // NW
CH = 64
NCHUNK = EPT // CH
TAIL = EPT - NCHUNK * CH
ROWW = 144
NP = 10000
RPT = NP // NS


def _qkv_body(x_ref, wq_ref, wk_ref, wv_ref, q_ref, k_ref, v_ref):
    x = x_ref[...]
    q_ref[...] = jnp.dot(x, wq_ref[...], preferred_element_type=jnp.float32)
    k_ref[...] = jnp.dot(x, wk_ref[...], preferred_element_type=jnp.float32)
    v_ref[...] = jnp.dot(x, wv_ref[...], preferred_element_type=jnp.float32)


_qkv_call = pl.pallas_call(
    _qkv_body,
    out_shape=[jax.ShapeDtypeStruct((N, D), jnp.float32)] * 3,
)


def _edge_pass(qrows, krows, vrows, svbuf, n_edges):
    lane = lax.iota(jnp.int32, 16)

    def edge_body(e, carry):
        wacc = jnp.zeros((16,), jnp.float32)
        for h in range(H):
            p = qrows[e, pl.ds(h * DH, DH)] * krows[e, pl.ds(h * DH, DH)]
            sh = jnp.sum(p, axis=0)
            wacc = jnp.where(lane == h, sh, wacc)
        w = jnp.exp(wacc * 0.25)
        svbuf[e, pl.ds(D, 16)] = jnp.where(lane < H, w, 0.0)
        for h in range(H):
            wb = w[jnp.full((16,), h, jnp.int32)]
            svbuf[e, pl.ds(h * DH, DH)] = vrows[e, pl.ds(h * DH, DH)] * wb
        return carry

    lax.fori_loop(0, n_edges, edge_body, 0)


def _sc_body(q_hbm, k_hbm, v_hbm, src_hbm, dst_hbm, z_hbm, out_hbm,
             didx, sidx, qrows, krows, vrows, svbuf, acc,
             semq, semk, semv):
    c = lax.axis_index("c")
    s = lax.axis_index("s")
    wid = c * NS + s

    pltpu.sync_copy(z_hbm, acc.at[pl.ds(s * RPT, RPT)])

    plsc.subcore_barrier()

    def do_chunk(base, ch):
        pltpu.sync_copy(dst_hbm.at[pl.ds(base, ch)], didx.at[pl.ds(0, ch)])
        pltpu.sync_copy(src_hbm.at[pl.ds(base, ch)], sidx.at[pl.ds(0, ch)])
        cq = pltpu.async_copy(q_hbm.at[didx.at[pl.ds(0, ch)]],
                              qrows.at[pl.ds(0, ch)], semq)
        ck = pltpu.async_copy(k_hbm.at[sidx.at[pl.ds(0, ch)]],
                              krows.at[pl.ds(0, ch)], semk)
        cv = pltpu.async_copy(v_hbm.at[sidx.at[pl.ds(0, ch)]],
                              vrows.at[pl.ds(0, ch)], semv)
        cq.wait()
        ck.wait()
        cv.wait()
        _edge_pass(qrows, krows, vrows, svbuf, ch)
        pltpu.sync_copy(svbuf.at[pl.ds(0, ch)],
                        acc.at[didx.at[pl.ds(0, ch)]], add=True)

    def chunk_body(i, carry):
        do_chunk(wid * EPT + i * CH, CH)
        return carry

    lax.fori_loop(0, NCHUNK, chunk_body, 0)
    do_chunk(wid * EPT + NCHUNK * CH, TAIL)

    plsc.subcore_barrier()
    pltpu.sync_copy(acc.at[pl.ds(s * RPT, RPT)],
                    out_hbm.at[c, pl.ds(s * RPT, RPT)])


_sc_call = functools.partial(
    pl.kernel,
    out_type=jax.ShapeDtypeStruct((NC, NP, ROWW), jnp.float32),
    mesh=plsc.VectorSubcoreMesh(core_axis_name="c", subcore_axis_name="s"),
    compiler_params=pltpu.CompilerParams(use_tc_tiling_on_sc=False,
                                         needs_layout_passes=False),
    scratch_types=[
        pltpu.VMEM((CH,), jnp.int32),
        pltpu.VMEM((CH,), jnp.int32),
        pltpu.VMEM((CH, D), jnp.float32),
        pltpu.VMEM((CH, D), jnp.float32),
        pltpu.VMEM((CH, D), jnp.float32),
        pltpu.VMEM((CH, ROWW), jnp.float32),
        pltpu.VMEM_SHARED((NP, ROWW), jnp.float32),
        pltpu.SemaphoreType.DMA,
        pltpu.SemaphoreType.DMA,
        pltpu.SemaphoreType.DMA,
    ],
)(_sc_body)


def _bn(y, g, b):
    m = jnp.mean(y, axis=0)
    d = y - m
    v = jnp.mean(d * d, axis=0)
    return g * d * lax.rsqrt(v + EPS) + b


def _epi_body(acc_ref, x_ref, wo_ref, g1_ref, b1_ref, w1_ref, w2_ref,
              g2_ref, b2_ref, out_ref):
    a = acc_ref[0] + acc_ref[1]
    num = a[:N, :D]
    den = a[:N, D:D + H]
    rep = (jax.lax.broadcasted_iota(jnp.int32, (H, D), 1) // DH
           == jax.lax.broadcasted_iota(jnp.int32, (H, D), 0)
           ).astype(jnp.float32)
    den_rep = jnp.dot(den, rep, preferred_element_type=jnp.float32)
    agg = num / (den_rep + 1e-20)
    attn = jnp.dot(agg, wo_ref[...], preferred_element_type=jnp.float32)
    h1 = _bn(attn + x_ref[...], g1_ref[...], b1_ref[...])
    f = jnp.dot(
        jnp.maximum(jnp.dot(h1, w1_ref[...], preferred_element_type=jnp.float32), 0.0),
        w2_ref[...], preferred_element_type=jnp.float32)
    out_ref[...] = _bn(h1 + f, g2_ref[...], b2_ref[...])


_epi_call = pl.pallas_call(
    _epi_body,
    out_shape=jax.ShapeDtypeStruct((N, D), jnp.float32),
)


def kernel(x, edge_index, Wq, Wk, Wv, Wo, gamma1, beta1, W1, W2, gamma2, beta2):
    src = edge_index[0]
    dst = edge_index[1]
    q, k, v = _qkv_call(x, Wq, Wk, Wv)
    zrows = jnp.zeros((RPT, ROWW), jnp.float32)
    acc2 = _sc_call(q, k, v, src, dst, zrows)
    return _epi_call(acc2, x, Wo, gamma1, beta1, W1, W2, gamma2, beta2)


# speedup vs baseline: 1.3048x; 1.3048x over previous
import functools

import jax
import jax.numpy as jnp
from jax import lax
from jax.experimental import pallas as pl
from jax.experimental.pallas import tpu as pltpu
from jax.experimental.pallas import tpu_sc as plsc

N = 10000
E = 320000
D = 128
H = 8
DH = 16
HID = 128
EPS = 1e-5

NC = 2
NS = 16
NW = NC * NS
EPT = E // NW
CH = 64
NCHUNK = EPT // CH
TAIL = EPT - NCHUNK * CH
ROWW = 144
NP = 10000
RPT = NP // NS


def _qkv_body(x_ref, wq_ref, wk_ref, wv_ref, q_ref, k_ref, v_ref):
    x = x_ref[...]
    q_ref[...] = jnp.dot(x, wq_ref[...], preferred_element_type=jnp.float32)
    k_ref[...] = jnp.dot(x, wk_ref[...], preferred_element_type=jnp.float32)
    v_ref[...] = jnp.dot(x, wv_ref[...], preferred_element_type=jnp.float32)


_qkv_call = pl.pallas_call(
    _qkv_body,
    out_shape=[jax.ShapeDtypeStruct((N, D), jnp.float32)] * 3,
)


def _edge_pass(qrows, krows, vrows, svbuf, n_edges):
    lane = lax.iota(jnp.int32, 16)

    def edge_body(e, carry):
        wacc = jnp.zeros((16,), jnp.float32)
        for h in range(H):
            p = qrows[e, pl.ds(h * DH, DH)] * krows[e, pl.ds(h * DH, DH)]
            sh = jnp.sum(p, axis=0)
            wacc = jnp.where(lane == h, sh, wacc)
        w = jnp.exp(wacc * 0.25)
        svbuf[e, pl.ds(D, 16)] = jnp.where(lane < H, w, 0.0)
        vv = [vrows[e, pl.ds(h * DH, DH)] for h in range(H)]
        wb = [w[jnp.full((16,), h, jnp.int32)] for h in range(H)]
        sv = [vv[h] * wb[h] for h in range(H)]
        for h in range(H):
            svbuf[e, pl.ds(h * DH, DH)] = sv[h]
        return carry

    lax.fori_loop(0, n_edges, edge_body, 0)


def _sc_body(q_hbm, k_hbm, v_hbm, src_hbm, dst_hbm, z_hbm, out_hbm,
             didx, sidx, qrows, krows, vrows, svbuf, acc,
             semq, semk, semv):
    c = lax.axis_index("c")
    s = lax.axis_index("s")
    wid = c * NS + s

    pltpu.sync_copy(z_hbm, acc.at[pl.ds(s * RPT, RPT)])

    plsc.subcore_barrier()

    def do_chunk(base, ch):
        pltpu.sync_copy(dst_hbm.at[pl.ds(base, ch)], didx.at[pl.ds(0, ch)])
        pltpu.sync_copy(src_hbm.at[pl.ds(base, ch)], sidx.at[pl.ds(0, ch)])
        cq = pltpu.async_copy(q_hbm.at[didx.at[pl.ds(0, ch)]],
                              qrows.at[pl.ds(0, ch)], semq)
        ck = pltpu.async_copy(k_hbm.at[sidx.at[pl.ds(0, ch)]],
                              krows.at[pl.ds(0, ch)], semk)
        cv = pltpu.async_copy(v_hbm.at[sidx.at[pl.ds(0, ch)]],
                              vrows.at[pl.ds(0, ch)], semv)
        cq.wait()
        ck.wait()
        cv.wait()
        _edge_pass(qrows, krows, vrows, svbuf, ch)
        pltpu.sync_copy(svbuf.at[pl.ds(0, ch)],
                        acc.at[didx.at[pl.ds(0, ch)]], add=True)

    def chunk_body(i, carry):
        do_chunk(wid * EPT + i * CH, CH)
        return carry

    lax.fori_loop(0, NCHUNK, chunk_body, 0)
    do_chunk(wid * EPT + NCHUNK * CH, TAIL)

    plsc.subcore_barrier()
    pltpu.sync_copy(acc.at[pl.ds(s * RPT, RPT)],
                    out_hbm.at[c, pl.ds(s * RPT, RPT)])


_sc_call = functools.partial(
    pl.kernel,
    out_type=jax.ShapeDtypeStruct((NC, NP, ROWW), jnp.float32),
    mesh=plsc.VectorSubcoreMesh(core_axis_name="c", subcore_axis_name="s"),
    compiler_params=pltpu.CompilerParams(use_tc_tiling_on_sc=False,
                                         needs_layout_passes=False),
    scratch_types=[
        pltpu.VMEM((CH,), jnp.int32),
        pltpu.VMEM((CH,), jnp.int32),
        pltpu.VMEM((CH, D), jnp.float32),
        pltpu.VMEM((CH, D), jnp.float32),
        pltpu.VMEM((CH, D), jnp.float32),
        pltpu.VMEM((CH, ROWW), jnp.float32),
        pltpu.VMEM_SHARED((NP, ROWW), jnp.float32),
        pltpu.SemaphoreType.DMA,
        pltpu.SemaphoreType.DMA,
        pltpu.SemaphoreType.DMA,
    ],
)(_sc_body)


def _bn(y, g, b):
    m = jnp.mean(y, axis=0)
    d = y - m
    v = jnp.mean(d * d, axis=0)
    return g * d * lax.rsqrt(v + EPS) + b


def _epi_body(acc_ref, x_ref, wo_ref, g1_ref, b1_ref, w1_ref, w2_ref,
              g2_ref, b2_ref, out_ref):
    a = acc_ref[0] + acc_ref[1]
    num = a[:N, :D]
    den = a[:N, D:D + H]
    rep = (jax.lax.broadcasted_iota(jnp.int32, (H, D), 1) // DH
           == jax.lax.broadcasted_iota(jnp.int32, (H, D), 0)
           ).astype(jnp.float32)
    den_rep = jnp.dot(den, rep, preferred_element_type=jnp.float32)
    agg = num / (den_rep + 1e-20)
    attn = jnp.dot(agg, wo_ref[...], preferred_element_type=jnp.float32)
    h1 = _bn(attn + x_ref[...], g1_ref[...], b1_ref[...])
    f = jnp.dot(
        jnp.maximum(jnp.dot(h1, w1_ref[...], preferred_element_type=jnp.float32), 0.0),
        w2_ref[...], preferred_element_type=jnp.float32)
    out_ref[...] = _bn(h1 + f, g2_ref[...], b2_ref[...])


_epi_call = pl.pallas_call(
    _epi_body,
    out_shape=jax.ShapeDtypeStruct((N, D), jnp.float32),
)


def kernel(x, edge_index, Wq, Wk, Wv, Wo, gamma1, beta1, W1, W2, gamma2, beta2):
    src = edge_index[0]
    dst = edge_index[1]
    q, k, v = _qkv_call(x, Wq, Wk, Wv)
    zrows = jnp.zeros((RPT, ROWW), jnp.float32)
    acc2 = _sc_call(q, k, v, src, dst, zrows)
    return _epi_call(acc2, x, Wo, gamma1, beta1, W1, W2, gamma2, beta2)


# speedup vs baseline: 1.5386x; 1.1792x over previous
import functools

import jax
import jax.numpy as jnp
from jax import lax
from jax.experimental import pallas as pl
from jax.experimental.pallas import tpu as pltpu
from jax.experimental.pallas import tpu_sc as plsc

N = 10000
E = 320000
D = 128
H = 8
DH = 16
HID = 128
EPS = 1e-5

NC = 2
NS = 16
NW = NC * NS
EPT = E // NW
CH = 64
NCHUNK = EPT // CH
TAIL = EPT - NCHUNK * CH
ROWW = 144
NP = 10000
RPT = NP // NS


def _qkv_body(x_ref, wq_ref, wk_ref, wv_ref, q_ref, k_ref, v_ref):
    x = x_ref[...]
    q_ref[...] = jnp.dot(x, wq_ref[...], preferred_element_type=jnp.float32)
    k_ref[...] = jnp.dot(x, wk_ref[...], preferred_element_type=jnp.float32)
    v_ref[...] = jnp.dot(x, wv_ref[...], preferred_element_type=jnp.float32)


_qkv_call = pl.pallas_call(
    _qkv_body,
    out_shape=[jax.ShapeDtypeStruct((N, D), jnp.float32)] * 3,
)


UN = 2


def _edge_pass(qrows, krows, vrows, svbuf, n_edges):
    lane = lax.iota(jnp.int32, 16)

    def edge_body(i, carry):
        es = [i * UN + u for u in range(UN)]
        ws = []
        for e in es:
            wacc = jnp.zeros((16,), jnp.float32)
            for h in range(H):
                p = qrows[e, pl.ds(h * DH, DH)] * krows[e, pl.ds(h * DH, DH)]
                sh = jnp.sum(p, axis=0)
                wacc = jnp.where(lane == h, sh, wacc)
            ws.append(jnp.exp(wacc * 0.25))
        for e, w in zip(es, ws):
            svbuf[e, pl.ds(D, 16)] = jnp.where(lane < H, w, 0.0)
            vv = [vrows[e, pl.ds(h * DH, DH)] for h in range(H)]
            wb = [w[jnp.full((16,), h, jnp.int32)] for h in range(H)]
            sv = [vv[h] * wb[h] for h in range(H)]
            for h in range(H):
                svbuf[e, pl.ds(h * DH, DH)] = sv[h]
        return carry

    lax.fori_loop(0, n_edges // UN, edge_body, 0)


def _sc_body(q_hbm, k_hbm, v_hbm, src_hbm, dst_hbm, z_hbm, out_hbm,
             didx, sidx, qrows, krows, vrows, svbuf, acc,
             semq, semk, semv):
    c = lax.axis_index("c")
    s = lax.axis_index("s")
    wid = c * NS + s

    pltpu.sync_copy(z_hbm, acc.at[pl.ds(s * RPT, RPT)])

    plsc.subcore_barrier()

    def do_chunk(base, ch):
        pltpu.sync_copy(dst_hbm.at[pl.ds(base, ch)], didx.at[pl.ds(0, ch)])
        pltpu.sync_copy(src_hbm.at[pl.ds(base, ch)], sidx.at[pl.ds(0, ch)])
        cq = pltpu.async_copy(q_hbm.at[didx.at[pl.ds(0, ch)]],
                              qrows.at[pl.ds(0, ch)], semq)
        ck = pltpu.async_copy(k_hbm.at[sidx.at[pl.ds(0, ch)]],
                              krows.at[pl.ds(0, ch)], semk)
        cv = pltpu.async_copy(v_hbm.at[sidx.at[pl.ds(0, ch)]],
                              vrows.at[pl.ds(0, ch)], semv)
        cq.wait()
        ck.wait()
        cv.wait()
        _edge_pass(qrows, krows, vrows, svbuf, ch)
        pltpu.sync_copy(svbuf.at[pl.ds(0, ch)],
                        acc.at[didx.at[pl.ds(0, ch)]], add=True)

    def chunk_body(i, carry):
        do_chunk(wid * EPT + i * CH, CH)
        return carry

    lax.fori_loop(0, NCHUNK, chunk_body, 0)
    do_chunk(wid * EPT + NCHUNK * CH, TAIL)

    plsc.subcore_barrier()
    pltpu.sync_copy(acc.at[pl.ds(s * RPT, RPT)],
                    out_hbm.at[c, pl.ds(s * RPT, RPT)])


_sc_call = functools.partial(
    pl.kernel,
    out_type=jax.ShapeDtypeStruct((NC, NP, ROWW), jnp.float32),
    mesh=plsc.VectorSubcoreMesh(core_axis_name="c", subcore_axis_name="s"),
    compiler_params=pltpu.CompilerParams(use_tc_tiling_on_sc=False,
                                         needs_layout_passes=False),
    scratch_types=[
        pltpu.VMEM((CH,), jnp.int32),
        pltpu.VMEM((CH,), jnp.int32),
        pltpu.VMEM((CH, D), jnp.float32),
        pltpu.VMEM((CH, D), jnp.float32),
        pltpu.VMEM((CH, D), jnp.float32),
        pltpu.VMEM((CH, ROWW), jnp.float32),
        pltpu.VMEM_SHARED((NP, ROWW), jnp.float32),
        pltpu.SemaphoreType.DMA,
        pltpu.SemaphoreType.DMA,
        pltpu.SemaphoreType.DMA,
    ],
)(_sc_body)


def _bn(y, g, b):
    m = jnp.mean(y, axis=0)
    d = y - m
    v = jnp.mean(d * d, axis=0)
    return g * d * lax.rsqrt(v + EPS) + b


def _epi_body(acc_ref, x_ref, wo_ref, g1_ref, b1_ref, w1_ref, w2_ref,
              g2_ref, b2_ref, out_ref):
    a = acc_ref[0] + acc_ref[1]
    num = a[:N, :D]
    den = a[:N, D:D + H]
    rep = (jax.lax.broadcasted_iota(jnp.int32, (H, D), 1) // DH
           == jax.lax.broadcasted_iota(jnp.int32, (H, D), 0)
           ).astype(jnp.float32)
    den_rep = jnp.dot(den, rep, preferred_element_type=jnp.float32)
    agg = num / (den_rep + 1e-20)
    attn = jnp.dot(agg, wo_ref[...], preferred_element_type=jnp.float32)
    h1 = _bn(attn + x_ref[...], g1_ref[...], b1_ref[...])
    f = jnp.dot(
        jnp.maximum(jnp.dot(h1, w1_ref[...], preferred_element_type=jnp.float32), 0.0),
        w2_ref[...], preferred_element_type=jnp.float32)
    out_ref[...] = _bn(h1 + f, g2_ref[...], b2_ref[...])


_epi_call = pl.pallas_call(
    _epi_body,
    out_shape=jax.ShapeDtypeStruct((N, D), jnp.float32),
)


def kernel(x, edge_index, Wq, Wk, Wv, Wo, gamma1, beta1, W1, W2, gamma2, beta2):
    src = edge_index[0]
    dst = edge_index[1]
    q, k, v = _qkv_call(x, Wq, Wk, Wv)
    zrows = jnp.zeros((RPT, ROWW), jnp.float32)
    acc2 = _sc_call(q, k, v, src, dst, zrows)
    return _epi_call(acc2, x, Wo, gamma1, beta1, W1, W2, gamma2, beta2)


# speedup vs baseline: 3.0023x; 1.9513x over previous
import functools

import jax
import jax.numpy as jnp
from jax import lax
from jax.experimental import pallas as pl
from jax.experimental.pallas import tpu as pltpu
from jax.experimental.pallas import tpu_sc as plsc

N = 10000
E = 320000
D = 128
H = 8
DH = 16
HID = 128
EPS = 1e-5

NC = 2
NS = 16
NW = NC * NS
EPT = E // NW
CH = 32
NCHUNK = EPT // CH
TAIL = EPT - NCHUNK * CH
ROWW = 144
NP = 10000
RPT = NP // NS


def _qkv_body(x_ref, wq_ref, wk_ref, wv_ref, q_ref, kv_ref):
    x = x_ref[...]
    q_ref[...] = jnp.dot(x, wq_ref[...], preferred_element_type=jnp.float32)
    kv_ref[:, :D] = jnp.dot(x, wk_ref[...], preferred_element_type=jnp.float32)
    kv_ref[:, D:] = jnp.dot(x, wv_ref[...], preferred_element_type=jnp.float32)


_qkv_call = pl.pallas_call(
    _qkv_body,
    out_shape=[jax.ShapeDtypeStruct((N, D), jnp.float32),
               jax.ShapeDtypeStruct((N, 2 * D), jnp.float32)],
)


UN = 2


def _edge_pass(qrows, kvrows, svbuf, n_edges):
    lane = lax.iota(jnp.int32, 16)

    def edge_body(i, carry):
        es = [i * UN + u for u in range(UN)]
        ws = []
        for e in es:
            wacc = jnp.zeros((16,), jnp.float32)
            for h in range(H):
                p = qrows[e, pl.ds(h * DH, DH)] * kvrows[e, pl.ds(h * DH, DH)]
                sh = jnp.sum(p, axis=0)
                wacc = jnp.where(lane == h, sh, wacc)
            ws.append(jnp.exp(wacc * 0.25))
        for e, w in zip(es, ws):
            svbuf[e, pl.ds(D, 16)] = jnp.where(lane < H, w, 0.0)
            vv = [kvrows[e, pl.ds(D + h * DH, DH)] for h in range(H)]
            wb = [w[jnp.full((16,), h, jnp.int32)] for h in range(H)]
            sv = [vv[h] * wb[h] for h in range(H)]
            for h in range(H):
                svbuf[e, pl.ds(h * DH, DH)] = sv[h]
        return carry

    lax.fori_loop(0, n_edges // UN, edge_body, 0)


def _sc_body(q_hbm, kv_hbm, src_hbm, dst_hbm, z_hbm, out_hbm,
             didx4, sidx4, qrows0, qrows1, kvrows0, kvrows1,
             svbuf0, svbuf1, tdidx, tsidx, acc,
             si0, si1, si2, si3, sq0, sq1, skv0, skv1, ssc0, ssc1):
    c = lax.axis_index("c")
    s = lax.axis_index("s")
    wid = c * NS + s
    ebase = wid * EPT
    qrows = [qrows0, qrows1]
    kvrows = [kvrows0, kvrows1]
    svbuf = [svbuf0, svbuf1]
    semidx = [si0, si1, si2, si3]
    semq = [sq0, sq1]
    semkv = [skv0, skv1]
    semsc = [ssc0, ssc1]

    pltpu.sync_copy(z_hbm, acc.at[pl.ds(s * RPT, RPT)])

    def fire_idx(jj, t):
        pltpu.async_copy(dst_hbm.at[pl.ds(ebase + jj * CH, CH)],
                         didx4.at[t], semidx[t])
        pltpu.async_copy(src_hbm.at[pl.ds(ebase + jj * CH, CH)],
                         sidx4.at[t], semidx[t])

    def wait_idx(t):
        pltpu.make_async_copy(dst_hbm.at[pl.ds(0, CH)], didx4.at[t],
                              semidx[t]).wait()
        pltpu.make_async_copy(src_hbm.at[pl.ds(0, CH)], sidx4.at[t],
                              semidx[t]).wait()

    def fire_gathers(b, t):
        pltpu.async_copy(q_hbm.at[didx4.at[t]], qrows[b], semq[b])
        pltpu.async_copy(kv_hbm.at[sidx4.at[t]], kvrows[b], semkv[b])

    def wait_gathers(b):
        pltpu.make_async_copy(q_hbm.at[didx4.at[0]], qrows[b], semq[b]).wait()
        pltpu.make_async_copy(kv_hbm.at[sidx4.at[0]], kvrows[b],
                              semkv[b]).wait()

    def fire_scatter(b, t):
        pltpu.async_copy(svbuf[b], acc.at[didx4.at[t]], semsc[b], add=True)

    def wait_scatter(b):
        pltpu.make_async_copy(svbuf[b], acc.at[didx4.at[0]], semsc[b]).wait()

    plsc.subcore_barrier()

    fire_idx(0, 0)
    wait_idx(0)
    fire_gathers(0, 0)
    fire_idx(1, 1)

    @pl.loop(0, NCHUNK, step=4)
    def _(j):
        for u in range(4):
            jj = j + u
            b = u % 2
            b1 = (u + 1) % 2
            t1 = (u + 1) % 4
            t2 = (u + 2) % 4
            if u < 3:
                wait_idx(t1)
                fire_gathers(b1, t1)
            else:
                @pl.when(jj + 1 < NCHUNK)
                def _():
                    wait_idx(t1)
                    fire_gathers(b1, t1)
            if u < 2:
                fire_idx(jj + 2, t2)
            else:
                @pl.when(jj + 2 < NCHUNK)
                def _():
                    fire_idx(jj + 2, t2)
            wait_gathers(b)
            _edge_pass(qrows[b], kvrows[b], svbuf[b], CH)
            if u >= 1:
                wait_scatter(b1)
            else:
                @pl.when(jj >= 1)
                def _():
                    wait_scatter(b1)
            fire_scatter(b, u)

    wait_scatter((NCHUNK - 1) % 2)

    tbase = ebase + NCHUNK * CH
    pltpu.sync_copy(dst_hbm.at[pl.ds(tbase, TAIL)], tdidx)
    pltpu.sync_copy(src_hbm.at[pl.ds(tbase, TAIL)], tsidx)
    cq = pltpu.async_copy(q_hbm.at[tdidx], qrows0.at[pl.ds(0, TAIL)], sq0)
    ckv = pltpu.async_copy(kv_hbm.at[tsidx], kvrows0.at[pl.ds(0, TAIL)], skv0)
    cq.wait()
    ckv.wait()
    _edge_pass(qrows0, kvrows0, svbuf0, TAIL)
    pltpu.sync_copy(svbuf0.at[pl.ds(0, TAIL)], acc.at[tdidx], add=True)

    plsc.subcore_barrier()
    pltpu.sync_copy(acc.at[pl.ds(s * RPT, RPT)],
                    out_hbm.at[c, pl.ds(s * RPT, RPT)])


_sc_call = functools.partial(
    pl.kernel,
    out_type=jax.ShapeDtypeStruct((NC, NP, ROWW), jnp.float32),
    mesh=plsc.VectorSubcoreMesh(core_axis_name="c", subcore_axis_name="s"),
    compiler_params=pltpu.CompilerParams(use_tc_tiling_on_sc=False,
                                         needs_layout_passes=False),
    scratch_types=[
        pltpu.VMEM((4, CH), jnp.int32),
        pltpu.VMEM((4, CH), jnp.int32),
        pltpu.VMEM((CH, D), jnp.float32),
        pltpu.VMEM((CH, D), jnp.float32),
        pltpu.VMEM((CH, 2 * D), jnp.float32),
        pltpu.VMEM((CH, 2 * D), jnp.float32),
        pltpu.VMEM((CH, ROWW), jnp.float32),
        pltpu.VMEM((CH, ROWW), jnp.float32),
        pltpu.VMEM((TAIL,), jnp.int32),
        pltpu.VMEM((TAIL,), jnp.int32),
        pltpu.VMEM_SHARED((NP, ROWW), jnp.float32),
        pltpu.SemaphoreType.DMA,
        pltpu.SemaphoreType.DMA,
        pltpu.SemaphoreType.DMA,
        pltpu.SemaphoreType.DMA,
        pltpu.SemaphoreType.DMA,
        pltpu.SemaphoreType.DMA,
        pltpu.SemaphoreType.DMA,
        pltpu.SemaphoreType.DMA,
        pltpu.SemaphoreType.DMA,
        pltpu.SemaphoreType.DMA,
    ],
)(_sc_body)


def _bn(y, g, b):
    m = jnp.mean(y, axis=0)
    d = y - m
    v = jnp.mean(d * d, axis=0)
    return g * d * lax.rsqrt(v + EPS) + b


def _epi_body(acc_ref, x_ref, wo_ref, g1_ref, b1_ref, w1_ref, w2_ref,
              g2_ref, b2_ref, out_ref):
    a = acc_ref[0] + acc_ref[1]
    num = a[:N, :D]
    den = a[:N, D:D + H]
    rep = (jax.lax.broadcasted_iota(jnp.int32, (H, D), 1) // DH
           == jax.lax.broadcasted_iota(jnp.int32, (H, D), 0)
           ).astype(jnp.float32)
    den_rep = jnp.dot(den, rep, preferred_element_type=jnp.float32)
    agg = num / (den_rep + 1e-20)
    attn = jnp.dot(agg, wo_ref[...], preferred_element_type=jnp.float32)
    h1 = _bn(attn + x_ref[...], g1_ref[...], b1_ref[...])
    f = jnp.dot(
        jnp.maximum(jnp.dot(h1, w1_ref[...], preferred_element_type=jnp.float32), 0.0),
        w2_ref[...], preferred_element_type=jnp.float32)
    out_ref[...] = _bn(h1 + f, g2_ref[...], b2_ref[...])


_epi_call = pl.pallas_call(
    _epi_body,
    out_shape=jax.ShapeDtypeStruct((N, D), jnp.float32),
)


def kernel(x, edge_index, Wq, Wk, Wv, Wo, gamma1, beta1, W1, W2, gamma2, beta2):
    src = edge_index[0]
    dst = edge_index[1]
    q, kv = _qkv_call(x, Wq, Wk, Wv)
    zrows = jnp.zeros((RPT, ROWW), jnp.float32)
    acc2 = _sc_call(q, kv, src, dst, zrows)
    return _epi_call(acc2, x, Wo, gamma1, beta1, W1, W2, gamma2, beta2)


# speedup vs baseline: 3.2027x; 1.0667x over previous
import functools

import jax
import jax.numpy as jnp
from jax import lax
from jax.experimental import pallas as pl
from jax.experimental.pallas import tpu as pltpu
from jax.experimental.pallas import tpu_sc as plsc

N = 10000
E = 320000
D = 128
H = 8
DH = 16
HID = 128
EPS = 1e-5

NC = 2
NS = 16
NW = NC * NS
EPT = E // NW
CH = 32
NCHUNK = EPT // CH
TAIL = EPT - NCHUNK * CH
ROWW = 144
NP = 10000
RPT = NP // NS


def _qkv_body(x_ref, wq_ref, wk_ref, wv_ref, q_ref, kv_ref):
    x = x_ref[...]
    q_ref[...] = jnp.dot(x, wq_ref[...], preferred_element_type=jnp.float32)
    kv_ref[:, :D] = jnp.dot(x, wk_ref[...], preferred_element_type=jnp.float32)
    kv_ref[:, D:] = jnp.dot(x, wv_ref[...], preferred_element_type=jnp.float32)


_qkv_call = pl.pallas_call(
    _qkv_body,
    out_shape=[jax.ShapeDtypeStruct((N, D), jnp.float32),
               jax.ShapeDtypeStruct((N, 2 * D), jnp.float32)],
)


UN = 4


def _edge_pass(qrows, kvrows, svbuf, n_edges):
    lane = lax.iota(jnp.int32, 16)

    def edge_body(i, carry):
        es = [i * UN + u for u in range(UN)]
        ws = []
        for e in es:
            wacc = jnp.zeros((16,), jnp.float32)
            for h in range(H):
                p = qrows[e, pl.ds(h * DH, DH)] * kvrows[e, pl.ds(h * DH, DH)]
                sh = jnp.sum(p, axis=0)
                wacc = jnp.where(lane == h, sh, wacc)
            ws.append(jnp.exp(wacc * 0.25))
        for e, w in zip(es, ws):
            svbuf[e, pl.ds(D, 16)] = jnp.where(lane < H, w, 0.0)
            vv = [kvrows[e, pl.ds(D + h * DH, DH)] for h in range(H)]
            wb = [w[jnp.full((16,), h, jnp.int32)] for h in range(H)]
            sv = [vv[h] * wb[h] for h in range(H)]
            for h in range(H):
                svbuf[e, pl.ds(h * DH, DH)] = sv[h]
        return carry

    lax.fori_loop(0, n_edges // UN, edge_body, 0)


def _sc_body(q_hbm, kv_hbm, src_hbm, dst_hbm, z_hbm, out_hbm,
             didx4, sidx4, qrows0, qrows1, kvrows0, kvrows1,
             svbuf0, svbuf1, tdidx, tsidx, acc,
             si0, si1, si2, si3, sq0, sq1, skv0, skv1, ssc0, ssc1):
    c = lax.axis_index("c")
    s = lax.axis_index("s")
    wid = c * NS + s
    ebase = wid * EPT
    qrows = [qrows0, qrows1]
    kvrows = [kvrows0, kvrows1]
    svbuf = [svbuf0, svbuf1]
    semidx = [si0, si1, si2, si3]
    semq = [sq0, sq1]
    semkv = [skv0, skv1]
    semsc = [ssc0, ssc1]

    pltpu.sync_copy(z_hbm, acc.at[pl.ds(s * RPT, RPT)])

    def fire_idx(jj, t):
        pltpu.async_copy(dst_hbm.at[pl.ds(ebase + jj * CH, CH)],
                         didx4.at[t], semidx[t])
        pltpu.async_copy(src_hbm.at[pl.ds(ebase + jj * CH, CH)],
                         sidx4.at[t], semidx[t])

    def wait_idx(t):
        pltpu.make_async_copy(dst_hbm.at[pl.ds(0, CH)], didx4.at[t],
                              semidx[t]).wait()
        pltpu.make_async_copy(src_hbm.at[pl.ds(0, CH)], sidx4.at[t],
                              semidx[t]).wait()

    def fire_gathers(b, t):
        pltpu.async_copy(q_hbm.at[didx4.at[t]], qrows[b], semq[b])
        pltpu.async_copy(kv_hbm.at[sidx4.at[t]], kvrows[b], semkv[b])

    def wait_gathers(b):
        pltpu.make_async_copy(q_hbm.at[didx4.at[0]], qrows[b], semq[b]).wait()
        pltpu.make_async_copy(kv_hbm.at[sidx4.at[0]], kvrows[b],
                              semkv[b]).wait()

    def fire_scatter(b, t):
        pltpu.async_copy(svbuf[b], acc.at[didx4.at[t]], semsc[b], add=True)

    def wait_scatter(b):
        pltpu.make_async_copy(svbuf[b], acc.at[didx4.at[0]], semsc[b]).wait()

    plsc.subcore_barrier()

    fire_idx(0, 0)
    wait_idx(0)
    fire_gathers(0, 0)
    fire_idx(1, 1)

    @pl.loop(0, NCHUNK, step=4)
    def _(j):
        for u in range(4):
            jj = j + u
            b = u % 2
            b1 = (u + 1) % 2
            t1 = (u + 1) % 4
            t2 = (u + 2) % 4
            if u < 3:
                wait_idx(t1)
                fire_gathers(b1, t1)
            else:
                @pl.when(jj + 1 < NCHUNK)
                def _():
                    wait_idx(t1)
                    fire_gathers(b1, t1)
            if u < 2:
                fire_idx(jj + 2, t2)
            else:
                @pl.when(jj + 2 < NCHUNK)
                def _():
                    fire_idx(jj + 2, t2)
            wait_gathers(b)
            _edge_pass(qrows[b], kvrows[b], svbuf[b], CH)
            if u >= 1:
                wait_scatter(b1)
            else:
                @pl.when(jj >= 1)
                def _():
                    wait_scatter(b1)
            fire_scatter(b, u)

    wait_scatter((NCHUNK - 1) % 2)

    tbase = ebase + NCHUNK * CH
    pltpu.sync_copy(dst_hbm.at[pl.ds(tbase, TAIL)], tdidx)
    pltpu.sync_copy(src_hbm.at[pl.ds(tbase, TAIL)], tsidx)
    cq = pltpu.async_copy(q_hbm.at[tdidx], qrows0.at[pl.ds(0, TAIL)], sq0)
    ckv = pltpu.async_copy(kv_hbm.at[tsidx], kvrows0.at[pl.ds(0, TAIL)], skv0)
    cq.wait()
    ckv.wait()
    _edge_pass(qrows0, kvrows0, svbuf0, TAIL)
    pltpu.sync_copy(svbuf0.at[pl.ds(0, TAIL)], acc.at[tdidx], add=True)

    plsc.subcore_barrier()
    pltpu.sync_copy(acc.at[pl.ds(s * RPT, RPT)],
                    out_hbm.at[c, pl.ds(s * RPT, RPT)])


_sc_call = functools.partial(
    pl.kernel,
    out_type=jax.ShapeDtypeStruct((NC, NP, ROWW), jnp.float32),
    mesh=plsc.VectorSubcoreMesh(core_axis_name="c", subcore_axis_name="s"),
    compiler_params=pltpu.CompilerParams(use_tc_tiling_on_sc=False,
                                         needs_layout_passes=False),
    scratch_types=[
        pltpu.VMEM((4, CH), jnp.int32),
        pltpu.VMEM((4, CH), jnp.int32),
        pltpu.VMEM((CH, D), jnp.float32),
        pltpu.VMEM((CH, D), jnp.float32),
        pltpu.VMEM((CH, 2 * D), jnp.float32),
        pltpu.VMEM((CH, 2 * D), jnp.float32),
        pltpu.VMEM((CH, ROWW), jnp.float32),
        pltpu.VMEM((CH, ROWW), jnp.float32),
        pltpu.VMEM((TAIL,), jnp.int32),
        pltpu.VMEM((TAIL,), jnp.int32),
        pltpu.VMEM_SHARED((NP, ROWW), jnp.float32),
        pltpu.SemaphoreType.DMA,
        pltpu.SemaphoreType.DMA,
        pltpu.SemaphoreType.DMA,
        pltpu.SemaphoreType.DMA,
        pltpu.SemaphoreType.DMA,
        pltpu.SemaphoreType.DMA,
        pltpu.SemaphoreType.DMA,
        pltpu.SemaphoreType.DMA,
        pltpu.SemaphoreType.DMA,
        pltpu.SemaphoreType.DMA,
    ],
)(_sc_body)


def _bn(y, g, b):
    m = jnp.mean(y, axis=0)
    d = y - m
    v = jnp.mean(d * d, axis=0)
    return g * d * lax.rsqrt(v + EPS) + b


def _epi_body(acc_ref, x_ref, wo_ref, g1_ref, b1_ref, w1_ref, w2_ref,
              g2_ref, b2_ref, out_ref):
    a = acc_ref[0] + acc_ref[1]
    num = a[:N, :D]
    den = a[:N, D:D + H]
    rep = (jax.lax.broadcasted_iota(jnp.int32, (H, D), 1) // DH
           == jax.lax.broadcasted_iota(jnp.int32, (H, D), 0)
           ).astype(jnp.float32)
    den_rep = jnp.dot(den, rep, preferred_element_type=jnp.float32)
    agg = num / (den_rep + 1e-20)
    attn = jnp.dot(agg, wo_ref[...], preferred_element_type=jnp.float32)
    h1 = _bn(attn + x_ref[...], g1_ref[...], b1_ref[...])
    f = jnp.dot(
        jnp.maximum(jnp.dot(h1, w1_ref[...], preferred_element_type=jnp.float32), 0.0),
        w2_ref[...], preferred_element_type=jnp.float32)
    out_ref[...] = _bn(h1 + f, g2_ref[...], b2_ref[...])


_epi_call = pl.pallas_call(
    _epi_body,
    out_shape=jax.ShapeDtypeStruct((N, D), jnp.float32),
)


def kernel(x, edge_index, Wq, Wk, Wv, Wo, gamma1, beta1, W1, W2, gamma2, beta2):
    src = edge_index[0]
    dst = edge_index[1]
    q, kv = _qkv_call(x, Wq, Wk, Wv)
    zrows = jnp.zeros((RPT, ROWW), jnp.float32)
    acc2 = _sc_call(q, kv, src, dst, zrows)
    return _epi_call(acc2, x, Wo, gamma1, beta1, W1, W2, gamma2, beta2)
